# baseline (speedup 1.0000x reference)
import functools

import jax
import jax.numpy as jnp
from jax import lax
from jax.experimental import pallas as pl
from jax.experimental.pallas import tpu as pltpu
from jax.experimental.pallas import tpu_sc as plsc

D_MODEL = 64
SCALE = float(D_MODEL) ** 0.5
NC = 2
NS = 16
NW = NC * NS
IB = 128


@functools.lru_cache(maxsize=None)
def _build(n_tok, n_pos, vocab):
    per_w = n_tok // NW
    n_blk = per_w // IB
    n_iter = n_pos * n_blk
    mesh = plsc.VectorSubcoreMesh(core_axis_name="c", subcore_axis_name="s")

    @functools.partial(
        pl.kernel,
        out_type=jax.ShapeDtypeStruct((n_pos, D_MODEL, n_tok), jnp.float32),
        mesh=mesh,
        scratch_types=[
            pltpu.VMEM((n_pos, per_w), jnp.int32),
            pltpu.VMEM((2, IB), jnp.int32),
            pltpu.VMEM((2, IB), jnp.int32),
            pltpu.VMEM((2, IB, 128), jnp.float32),
            pltpu.VMEM((2, D_MODEL, IB), jnp.float32),
            pltpu.SemaphoreType.DMA,
            pltpu.SemaphoreType.DMA,
            pltpu.SemaphoreType.DMA,
            pltpu.SemaphoreType.DMA,
        ],
        compiler_params=pltpu.CompilerParams(needs_layout_passes=False),
    )
    def emb(xt_hbm, t128_hbm, out_hbm, xi_v, idx_v, h64_v, g_v, o_v,
            sem_g0, sem_g1, sem_s0, sem_s1):
        wid = lax.axis_index("s") * NC + lax.axis_index("c")
        i0 = wid * per_w
        sems_g = (sem_g0, sem_g1)
        sems_s = (sem_s0, sem_s1)

        pltpu.sync_copy(xt_hbm.at[:, pl.ds(i0, per_w)], xi_v)

        def prep_and_fire(k, b):
            t = k >> 2
            ib = (k & 3) * IB
            for g in range(IB // 16):
                sl = pl.ds(g * 16, 16)
                raw = xi_v[t, pl.ds(ib + g * 16, 16)]
                h64_v[b, sl] = (raw & 1) << 6
                idx_v[b, sl] = lax.shift_right_logical(raw, 1)
            pltpu.async_copy(t128_hbm.at[idx_v.at[b]], g_v.at[b], sems_g[b])

        def wait_gather(b):
            pltpu.make_async_copy(
                t128_hbm.at[pl.ds(0, IB)], g_v.at[b], sems_g[b]
            ).wait()

        def wait_store(b):
            pltpu.make_async_copy(
                o_v.at[b], out_hbm.at[0, :, pl.ds(0, IB)], sems_s[b]
            ).wait()

        def process(k, b):
            t = k >> 2
            ib = i0 + (k & 3) * IB
            lane = lax.iota(jnp.int32, 16)
            rows_g = [lane + g * 16 for g in range(IB // 16)]
            hg_g = [h64_v[b, pl.ds(g * 16, 16)] for g in range(IB // 16)]
            gb = g_v.at[b]
            ob = o_v.at[b]

            @plsc.parallel_loop(0, D_MODEL // 8, unroll=2)
            def cbody(c8):
                for cc in range(8):
                    d = (lane + (c8 * 8 + cc)) & (D_MODEL - 1)
                    for g in range(IB // 16):
                        col = hg_g[g] + d
                        v = plsc.load_gather(gb, [rows_g[g], col])
                        plsc.store_scatter(ob, [d, rows_g[g]], v * SCALE)

            pltpu.async_copy(
                o_v.at[b], out_hbm.at[t, :, pl.ds(ib, IB)], sems_s[b]
            )

        prep_and_fire(0, 0)

        def kbody(k2, _):
            for b in range(2):
                k = k2 * 2 + b

                @pl.when(k + 1 < n_iter)
                def _():
                    prep_and_fire(k + 1, 1 - b)

                wait_gather(b)

                @pl.when(k >= 2)
                def _():
                    wait_store(b)

                process(k, b)
            return 0

        lax.fori_loop(0, n_iter // 2, kbody, 0)
        wait_store(0)
        wait_store(1)

    return emb


def kernel(x, table):
    n_seq, n_pos = x.shape
    xt = x.T.astype(jnp.int32)
    t128 = table.reshape(table.shape[0] // 2, 128)
    out = _build(n_seq, n_pos, table.shape[0])(xt, t128)
    return out.transpose(2, 0, 1)


# speedup vs baseline: 1.1111x; 1.1111x over previous
import functools

import jax
import jax.numpy as jnp
from jax import lax
from jax.experimental import pallas as pl
from jax.experimental.pallas import tpu as pltpu
from jax.experimental.pallas import tpu_sc as plsc

D_MODEL = 64
SCALE = float(D_MODEL) ** 0.5
NC = 2
NS = 16
NW = NC * NS
IB = 128


@functools.lru_cache(maxsize=None)
def _build(n_tok, n_pos, vocab):
    per_w = n_tok // NW
    n_blk = per_w // IB
    n_iter = n_pos * n_blk
    mesh = plsc.VectorSubcoreMesh(core_axis_name="c", subcore_axis_name="s")

    @functools.partial(
        pl.kernel,
        out_type=jax.ShapeDtypeStruct((n_pos, D_MODEL, n_tok), jnp.float32),
        mesh=mesh,
        scratch_types=[
            pltpu.VMEM((n_pos, per_w), jnp.int32),
            pltpu.VMEM((2, IB), jnp.int32),
            pltpu.VMEM((2, IB), jnp.int32),
            pltpu.VMEM((2, IB, 128), jnp.float32),
            pltpu.VMEM((2, D_MODEL, IB), jnp.float32),
            pltpu.SemaphoreType.DMA,
            pltpu.SemaphoreType.DMA,
            pltpu.SemaphoreType.DMA,
            pltpu.SemaphoreType.DMA,
        ],
        compiler_params=pltpu.CompilerParams(needs_layout_passes=False),
    )
    def emb(xt_hbm, t128_hbm, out_hbm, xi_v, idx_v, h64_v, g_v, o_v,
            sem_g0, sem_g1, sem_s0, sem_s1):
        wid = lax.axis_index("s") * NC + lax.axis_index("c")
        i0 = wid * per_w
        sems_g = (sem_g0, sem_g1)
        sems_s = (sem_s0, sem_s1)

        pltpu.sync_copy(xt_hbm.at[:, pl.ds(i0, per_w)], xi_v)

        def prep_and_fire(k, b):
            t = k >> 2
            ib = (k & 3) * IB
            for g in range(IB // 16):
                sl = pl.ds(g * 16, 16)
                raw = xi_v[t, pl.ds(ib + g * 16, 16)]
                h64_v[b, sl] = (raw & 1) << 6
                idx_v[b, sl] = lax.shift_right_logical(raw, 1)
            pltpu.async_copy(t128_hbm.at[idx_v.at[b]], g_v.at[b], sems_g[b])

        def wait_gather(b):
            pltpu.make_async_copy(
                t128_hbm.at[pl.ds(0, IB)], g_v.at[b], sems_g[b]
            ).wait()

        def wait_store(b):
            pltpu.make_async_copy(
                o_v.at[b], out_hbm.at[0, :, pl.ds(0, IB)], sems_s[b]
            ).wait()

        def process(k, b):
            t = k >> 2
            ib = i0 + (k & 3) * IB
            lane = lax.iota(jnp.int32, 16)
            rows_g = [lane + g * 16 for g in range(IB // 16)]
            hg_g = [h64_v[b, pl.ds(g * 16, 16)] for g in range(IB // 16)]
            gb = g_v.at[b]
            ob = o_v.at[b]

            def cbody(c8, _):
                for cc in range(8):
                    d = (lane + (c8 * 8 + cc)) & (D_MODEL - 1)
                    for g in range(IB // 16):
                        col = hg_g[g] + d
                        v = plsc.load_gather(gb, [rows_g[g], col])
                        plsc.store_scatter(ob, [d, rows_g[g]], v * SCALE)
                return 0

            lax.fori_loop(0, D_MODEL // 8, cbody, 0)
            pltpu.async_copy(
                o_v.at[b], out_hbm.at[t, :, pl.ds(ib, IB)], sems_s[b]
            )

        prep_and_fire(0, 0)

        def kbody(k2, _):
            for b in range(2):
                k = k2 * 2 + b

                @pl.when(k + 1 < n_iter)
                def _():
                    prep_and_fire(k + 1, 1 - b)

                wait_gather(b)

                @pl.when(k >= 2)
                def _():
                    wait_store(b)

                process(k, b)
            return 0

        lax.fori_loop(0, n_iter // 2, kbody, 0)
        wait_store(0)
        wait_store(1)

    return emb


def kernel(x, table):
    n_seq, n_pos = x.shape
    xt = x.T.astype(jnp.int32)
    t128 = table.reshape(table.shape[0] // 2, 128)
    out = _build(n_seq, n_pos, table.shape[0])(xt, t128)
    return out.transpose(2, 0, 1)


# speedup vs baseline: 1.1142x; 1.0028x over previous
import functools

import jax
import jax.numpy as jnp
from jax import lax
from jax.experimental import pallas as pl
from jax.experimental.pallas import tpu as pltpu
from jax.experimental.pallas import tpu_sc as plsc

D_MODEL = 64
SCALE = float(D_MODEL) ** 0.5
NC = 2
NS = 16
NW = NC * NS
IB = 128


@functools.lru_cache(maxsize=None)
def _build(n_tok, n_pos, vocab):
    per_w = n_tok // NW
    n_blk = per_w // IB
    n_iter = n_pos * n_blk
    mesh = plsc.VectorSubcoreMesh(core_axis_name="c", subcore_axis_name="s")

    @functools.partial(
        pl.kernel,
        out_type=jax.ShapeDtypeStruct((n_pos, D_MODEL, n_tok), jnp.float32),
        mesh=mesh,
        scratch_types=[
            pltpu.VMEM((n_pos, per_w), jnp.int32),
            pltpu.VMEM((2, IB), jnp.int32),
            pltpu.VMEM((2, IB), jnp.int32),
            pltpu.VMEM((2, IB, 128), jnp.float32),
            pltpu.VMEM((2, D_MODEL, IB), jnp.float32),
            pltpu.SemaphoreType.DMA,
            pltpu.SemaphoreType.DMA,
            pltpu.SemaphoreType.DMA,
            pltpu.SemaphoreType.DMA,
        ],
        compiler_params=pltpu.CompilerParams(needs_layout_passes=False),
    )
    def emb(xt_hbm, t128_hbm, out_hbm, xi_v, idx_v, h64_v, g_v, o_v,
            sem_g0, sem_g1, sem_s0, sem_s1):
        wid = lax.axis_index("s") * NC + lax.axis_index("c")
        i0 = wid * per_w
        sems_g = (sem_g0, sem_g1)
        sems_s = (sem_s0, sem_s1)

        pltpu.sync_copy(xt_hbm.at[:, pl.ds(i0, per_w)], xi_v)

        def prep_and_fire(k, b):
            t = k >> 2
            ib = (k & 3) * IB
            for g in range(IB // 16):
                sl = pl.ds(g * 16, 16)
                raw = xi_v[t, pl.ds(ib + g * 16, 16)]
                h64_v[b, sl] = (raw & 1) << 6
                idx_v[b, sl] = lax.shift_right_logical(raw, 1)
            pltpu.async_copy(t128_hbm.at[idx_v.at[b]], g_v.at[b], sems_g[b])

        def wait_gather(b):
            pltpu.make_async_copy(
                t128_hbm.at[pl.ds(0, IB)], g_v.at[b], sems_g[b]
            ).wait()

        def wait_store(b):
            pltpu.make_async_copy(
                o_v.at[b], out_hbm.at[0, :, pl.ds(0, IB)], sems_s[b]
            ).wait()

        def process(k, b):
            t = k >> 2
            ib = i0 + (k & 3) * IB
            lane = lax.iota(jnp.int32, 16)
            rows_g = [lane + g * 16 for g in range(IB // 16)]
            hg_g = [h64_v[b, pl.ds(g * 16, 16)] for g in range(IB // 16)]
            gb = g_v.at[b]
            ob = o_v.at[b]

            def cbody(c16, _):
                for cc in range(16):
                    d = (lane + (c16 * 16 + cc)) & (D_MODEL - 1)
                    for g in range(IB // 16):
                        col = hg_g[g] + d
                        v = plsc.load_gather(gb, [rows_g[g], col])
                        plsc.store_scatter(ob, [d, rows_g[g]], v * SCALE)
                return 0

            lax.fori_loop(0, D_MODEL // 16, cbody, 0)
            pltpu.async_copy(
                o_v.at[b], out_hbm.at[t, :, pl.ds(ib, IB)], sems_s[b]
            )

        prep_and_fire(0, 0)

        def kbody(k2, _):
            for b in range(2):
                k = k2 * 2 + b

                @pl.when(k + 1 < n_iter)
                def _():
                    prep_and_fire(k + 1, 1 - b)

                wait_gather(b)

                @pl.when(k >= 2)
                def _():
                    wait_store(b)

                process(k, b)
            return 0

        lax.fori_loop(0, n_iter // 2, kbody, 0)
        wait_store(0)
        wait_store(1)

    return emb


def kernel(x, table):
    n_seq, n_pos = x.shape
    xt = x.T.astype(jnp.int32)
    t128 = table.reshape(table.shape[0] // 2, 128)
    out = _build(n_seq, n_pos, table.shape[0])(xt, t128)
    return out.transpose(2, 0, 1)
